# baseline (speedup 1.0000x reference)
import functools

import jax
import jax.numpy as jnp
from jax import lax
from jax.experimental import pallas as pl
from jax.experimental.pallas import tpu as pltpu
from jax.experimental.pallas import tpu_sc as plsc

_ROWS = 16384
_NCOL = 128
_OCOL = 129
_NC = 2
_NS = 16
_NW = _NC * _NS
_RPW = _ROWS // _NW
_CH = 64
_NCH = _RPW // _CH
_NBUF = 7


def _compute_tail(buf):
    for j in range(_CH // 16):
        rows = lax.iota(jnp.int32, 16) + j * 16
        b = plsc.load_gather(buf, [rows, jnp.full((16,), 124, jnp.int32)])
        c = plsc.load_gather(buf, [rows, jnp.full((16,), 125, jnp.int32)])
        l = plsc.load_gather(buf, [rows, jnp.full((16,), 126, jnp.int32)])
        g = plsc.load_gather(buf, [rows, jnp.full((16,), 127, jnp.int32)])
        r1 = c / (c + b)
        r2 = c / (c + l + g)
        r3 = g / (g + l)
        plsc.store_scatter(buf, [rows, jnp.full((16,), 126, jnp.int32)], r1)
        plsc.store_scatter(buf, [rows, jnp.full((16,), 127, jnp.int32)], r2)
        plsc.store_scatter(buf, [rows, jnp.full((16,), 128, jnp.int32)], r3)


@functools.partial(
    pl.kernel,
    out_type=jax.ShapeDtypeStruct((_ROWS, _OCOL), jnp.float32),
    mesh=plsc.VectorSubcoreMesh(
        core_axis_name="c", subcore_axis_name="s",
        num_cores=_NC, num_subcores=_NS),
    scratch_types=(
        [pltpu.VMEM((_CH, _OCOL), jnp.float32)] * _NBUF
        + [pltpu.SemaphoreType.DMA] * (2 * _NBUF)
    ),
    compiler_params=pltpu.CompilerParams(
        use_tc_tiling_on_sc=True, needs_layout_passes=False),
)
def _sc_kernel(x_hbm, out_hbm, *scr):
    bufs = scr[:_NBUF]
    sin = scr[_NBUF:2 * _NBUF]
    sout = scr[2 * _NBUF:3 * _NBUF]
    wid = lax.axis_index("s") * _NC + lax.axis_index("c")
    base = wid * _RPW

    def start_in(i):
        r0 = base + i * _CH
        return pltpu.make_async_copy(
            x_hbm.at[pl.ds(r0, _CH)], bufs[i % _NBUF].at[:, pl.ds(0, _NCOL)],
            sin[i % _NBUF])

    def start_out(i):
        r0 = base + i * _CH
        return pltpu.make_async_copy(
            bufs[i % _NBUF], out_hbm.at[pl.ds(r0, _CH)], sout[i % _NBUF])

    h_in = {}
    h_out = {}
    for i in range(min(_NBUF, _NCH)):
        h_in[i] = start_in(i)
        h_in[i].start()
    for i in range(_NCH):
        h_in[i].wait()
        _compute_tail(bufs[i % _NBUF])
        h_out[i] = start_out(i)
        h_out[i].start()
        nxt = i - 2 + _NBUF
        if i >= 2 and nxt < _NCH:
            h_out[i - 2].wait()
            h_in[nxt] = start_in(nxt)
            h_in[nxt].start()
    for i in range(max(0, _NCH - _NBUF), _NCH):
        h_out[i].wait()


def kernel(x):
    return _sc_kernel(x)


# speedup vs baseline: 1.0065x; 1.0065x over previous
import functools

import jax
import jax.numpy as jnp
from jax import lax
from jax.experimental import pallas as pl
from jax.experimental.pallas import tpu as pltpu
from jax.experimental.pallas import tpu_sc as plsc

_ROWS = 16384
_NCOL = 128
_OCOL = 129
_NC = 2
_NS = 16
_NW = _NC * _NS
_RPW = _ROWS // _NW
_CH = 64
_NCH = _RPW // _CH
_NBUF = 7


def _compute_tail(buf):
    for j in range(_CH // 16):
        rows = lax.iota(jnp.int32, 16) + j * 16
        b = plsc.load_gather(buf, [rows, jnp.full((16,), 124, jnp.int32)])
        c = plsc.load_gather(buf, [rows, jnp.full((16,), 125, jnp.int32)])
        l = plsc.load_gather(buf, [rows, jnp.full((16,), 126, jnp.int32)])
        g = plsc.load_gather(buf, [rows, jnp.full((16,), 127, jnp.int32)])
        r1 = c / (c + b)
        r2 = c / (c + l + g)
        r3 = g / (g + l)
        plsc.store_scatter(buf, [rows, jnp.full((16,), 126, jnp.int32)], r1)
        plsc.store_scatter(buf, [rows, jnp.full((16,), 127, jnp.int32)], r2)
        plsc.store_scatter(buf, [rows, jnp.full((16,), 128, jnp.int32)], r3)


@functools.partial(
    pl.kernel,
    out_type=jax.ShapeDtypeStruct((_ROWS, _OCOL), jnp.float32),
    mesh=plsc.VectorSubcoreMesh(
        core_axis_name="c", subcore_axis_name="s",
        num_cores=_NC, num_subcores=_NS),
    scratch_types=(
        [pltpu.VMEM((_CH, _OCOL), jnp.float32)] * _NBUF
        + [pltpu.SemaphoreType.DMA] * (2 * _NBUF)
    ),
    compiler_params=pltpu.CompilerParams(
        use_tc_tiling_on_sc=True, needs_layout_passes=False),
)
def _sc_kernel(x_hbm, out_hbm, *scr):
    bufs = scr[:_NBUF]
    sin = scr[_NBUF:2 * _NBUF]
    sout = scr[2 * _NBUF:3 * _NBUF]
    wid = lax.axis_index("s") * _NC + lax.axis_index("c")
    base = wid * _RPW

    def start_in(i):
        r0 = base + i * _CH
        return pltpu.make_async_copy(
            x_hbm.at[pl.ds(r0, _CH)], bufs[i % _NBUF].at[:, pl.ds(0, _NCOL)],
            sin[i % _NBUF])

    def start_out(i):
        r0 = base + i * _CH
        return pltpu.make_async_copy(
            bufs[i % _NBUF], out_hbm.at[pl.ds(r0, _CH)], sout[i % _NBUF])

    h_in = {}
    h_out = {}
    for i in range(min(_NBUF, _NCH)):
        h_in[i] = start_in(i)
        h_in[i].start()
    for i in range(_NCH):
        h_in[i].wait()
        _compute_tail(bufs[i % _NBUF])
        h_out[i] = start_out(i)
        h_out[i].start()
        nxt = i - 1 + _NBUF
        if i >= 1 and nxt < _NCH:
            h_out[i - 1].wait()
            h_in[nxt] = start_in(nxt)
            h_in[nxt].start()
    for i in range(max(0, _NCH - _NBUF), _NCH):
        h_out[i].wait()


def kernel(x):
    return _sc_kernel(x)
